# Initial kernel scaffold; baseline (speedup 1.0000x reference)
import jax
import jax.numpy as jnp
from jax import lax
from jax.experimental import pallas as pl
from jax.experimental.pallas import tpu as pltpu
from jax.experimental.pallas import tpu_sc as plsc

N = 50000
D = 512
E = 800000
G = 512
OUT = 7
_BN_SCALE = 1.0 / jnp.sqrt(jnp.float32(1.0 + 1e-05))

R = 1568
NB = 32
NP = R * NB
NW = 32
CHW = 128
CH = 200
EPT = CH * CHW
EP = EPT * NW
RPT = NP // 16
RC = RPT // 4


def _pre_body(x_ref, wa_ref, ba_ref, wb_ref, bb_ref, out_ref):
    i = pl.program_id(0)
    xb = x_ref[...]
    t = jnp.maximum(jnp.dot(xb, wa_ref[...], preferred_element_type=jnp.float32)
                    + ba_ref[...], 0.0)
    nx = jnp.maximum(jnp.dot(t, wb_ref[...], preferred_element_type=jnp.float32)
                     + bb_ref[...], 0.0)
    rows = lax.broadcasted_iota(jnp.int32, (R, 64), 0) + i * R
    nx = jnp.where(rows < N, nx, 0.0)
    out_ref[0] = nx[:, :32]
    out_ref[1] = nx[:, 32:]


def _pre_call(x, wa, ba, wb, bb):
    return pl.pallas_call(
        _pre_body,
        grid=(NB,),
        in_specs=[
            pl.BlockSpec((R, D), lambda i: (i, 0)),
            pl.BlockSpec((D, 32), lambda i: (0, 0)),
            pl.BlockSpec((1, 32), lambda i: (0, 0)),
            pl.BlockSpec((32, 64), lambda i: (0, 0)),
            pl.BlockSpec((1, 64), lambda i: (0, 0)),
        ],
        out_specs=pl.BlockSpec((2, R, 32), lambda i: (0, i, 0)),
        out_shape=jax.ShapeDtypeStruct((2, NP, 32), jnp.float32),
    )(x, wa, ba, wb, bb)


def _seg_body(table_hbm, srcidx_hbm, dstidx_hbm, out_hbm,
              src_v, dst_v, rows_v, stage_v, acc_sh, sem):
    c = lax.axis_index("c")
    s = lax.axis_index("s")
    w = c * 16 + s

    pltpu.sync_copy(srcidx_hbm.at[w], src_v)
    pltpu.sync_copy(dstidx_hbm.at[s], dst_v)

    zv = jnp.zeros((16,), jnp.float32)

    def _zero(r, carry):
        stage_v[r, pl.ds(0, 16)] = zv
        stage_v[r, pl.ds(16, 16)] = zv
        return carry

    lax.fori_loop(0, RC, _zero, 0)
    base = s * RPT
    for k in range(4):
        pltpu.sync_copy(stage_v, acc_sh.at[pl.ds(base + k * RC, RC)])
    plsc.subcore_barrier()

    def _edge(j, carry):
        pltpu.async_copy(table_hbm.at[src_v.at[j]], rows_v, sem).wait()
        pltpu.sync_copy(rows_v, acc_sh.at[dst_v.at[j]], add=True)
        return carry

    lax.fori_loop(0, CH, _edge, 0)
    plsc.subcore_barrier()

    obase = c * NP + base
    for k in range(4):
        pltpu.sync_copy(acc_sh.at[pl.ds(base + k * RC, RC)], stage_v)
        pltpu.sync_copy(stage_v, out_hbm.at[pl.ds(obase + k * RC, RC)])


def _seg_call(table2, srcidx, dstidx):
    table_flat = table2.reshape(2 * NP, 32)
    mesh = plsc.VectorSubcoreMesh(core_axis_name="c", subcore_axis_name="s")
    out = pl.kernel(
        _seg_body,
        out_type=jax.ShapeDtypeStruct((2 * NP, 32), jnp.float32),
        mesh=mesh,
        scratch_types=[
            pltpu.VMEM((CH, CHW), jnp.int32),
            pltpu.VMEM((CH, CHW), jnp.int32),
            pltpu.VMEM((CHW, 32), jnp.float32),
            pltpu.VMEM((RC, 32), jnp.float32),
            pltpu.VMEM_SHARED((NP, 32), jnp.float32),
            pltpu.SemaphoreType.DMA,
        ],
    )(table_flat, srcidx, dstidx)
    return out.reshape(2, NP, 32)


def _gin0_body(nx_ref, agg_ref, w1_ref, a1_ref, c1_ref, w2_ref, b2_ref,
               out_ref):
    nxb = jnp.concatenate([nx_ref[0], nx_ref[1]], axis=1)
    aggb = jnp.concatenate([agg_ref[0], agg_ref[1]], axis=1)
    h = nxb + aggb
    t = jnp.maximum(jnp.dot(h, w1_ref[...], preferred_element_type=jnp.float32)
                    * a1_ref[...] + c1_ref[...], 0.0)
    g0 = (jnp.dot(t, w2_ref[...], preferred_element_type=jnp.float32)
          + b2_ref[...] + nxb)
    out_ref[0] = g0[:, :32]
    out_ref[1] = g0[:, 32:]


def _gin0_call(nx, agg, w1, a1, c1, w2, b2):
    spec2 = pl.BlockSpec((2, R, 32), lambda i: (0, i, 0))
    return pl.pallas_call(
        _gin0_body,
        grid=(NB,),
        in_specs=[
            spec2, spec2,
            pl.BlockSpec((64, 64), lambda i: (0, 0)),
            pl.BlockSpec((1, 64), lambda i: (0, 0)),
            pl.BlockSpec((1, 64), lambda i: (0, 0)),
            pl.BlockSpec((64, 64), lambda i: (0, 0)),
            pl.BlockSpec((1, 64), lambda i: (0, 0)),
        ],
        out_specs=spec2,
        out_shape=jax.ShapeDtypeStruct((2, NP, 32), jnp.float32),
    )(nx, agg, w1, a1, c1, w2, b2)


def _fin_body(nx_ref, g0_ref, agg_ref, batch_ref, w1_ref, a1_ref, c1_ref,
              w2_ref, b2_ref, wp1_ref, bp1_ref, wp2_ref, bp2_ref,
              out_ref, acc_ref):
    i = pl.program_id(0)
    nxb = jnp.concatenate([nx_ref[0], nx_ref[1]], axis=1)
    g0b = jnp.concatenate([g0_ref[0], g0_ref[1]], axis=1)
    aggb = jnp.concatenate([agg_ref[0], agg_ref[1]], axis=1)
    h = g0b + aggb
    t = jnp.maximum(jnp.dot(h, w1_ref[...], preferred_element_type=jnp.float32)
                    * a1_ref[...] + c1_ref[...], 0.0)
    g1 = (jnp.dot(t, w2_ref[...], preferred_element_type=jnp.float32)
          + b2_ref[...] + g0b + nxb)

    bb = batch_ref[0]
    onehot = (lax.broadcasted_iota(jnp.int32, (G, R), 0) == bb).astype(
        jnp.float32)
    gpad = jnp.concatenate(
        [g1, jnp.ones((R, 1), jnp.float32), jnp.zeros((R, 63), jnp.float32)],
        axis=1)
    contrib = jnp.dot(onehot, gpad, preferred_element_type=jnp.float32)

    @pl.when(i == 0)
    def _():
        acc_ref[...] = jnp.zeros((G, 128), jnp.float32)

    acc_ref[...] += contrib

    @pl.when(i == NB - 1)
    def _():
        a = acc_ref[...]
        mean = a[:, :64] / jnp.maximum(a[:, 64:65], 1.0)
        t2 = jnp.maximum(
            jnp.dot(mean, wp1_ref[...], preferred_element_type=jnp.float32)
            + bp1_ref[...], 0.0)
        o = (jnp.dot(t2, wp2_ref[...], preferred_element_type=jnp.float32)
             + bp2_ref[...])
        m = jnp.max(o, axis=1, keepdims=True)
        lse = m + jnp.log(jnp.sum(jnp.exp(o - m), axis=1, keepdims=True))
        out_ref[...] = o - lse


def _fin_call(nx, g0, agg, batch3, w1, a1, c1, w2, b2, wp1, bp1, wp2, bp2):
    spec2 = pl.BlockSpec((2, R, 32), lambda i: (0, i, 0))
    wfull = lambda r, c: pl.BlockSpec((r, c), lambda i: (0, 0))
    return pl.pallas_call(
        _fin_body,
        grid=(NB,),
        in_specs=[
            spec2, spec2, spec2,
            pl.BlockSpec((1, 1, R), lambda i: (i, 0, 0)),
            wfull(64, 64), wfull(1, 64), wfull(1, 64), wfull(64, 64),
            wfull(1, 64), wfull(64, 32), wfull(1, 32), wfull(32, OUT),
            wfull(1, OUT),
        ],
        out_specs=pl.BlockSpec((G, OUT), lambda i: (0, 0)),
        out_shape=jax.ShapeDtypeStruct((G, OUT), jnp.float32),
        scratch_shapes=[pltpu.VMEM((G, 128), jnp.float32)],
    )(nx, g0, agg, batch3, w1, a1, c1, w2, b2, wp1, bp1, wp2, bp2)


def kernel(x, edge_index, batch,
           W_pre1, b_pre1, W_pre2, b_pre2, W_pre3, b_pre3, W_pre4, b_pre4,
           gin0_W1, gin0_b1, gin0_g, gin0_bb, gin0_W2, gin0_b2, bn0_g, bn0_b,
           gin1_W1, gin1_b1, gin1_g, gin1_bb, gin1_W2, gin1_b2, bn1_g, bn1_b,
           W_post1, b_post1, W_post2, b_post2):
    f32 = jnp.float32
    s = _BN_SCALE

    wa = jnp.zeros((D, 32), f32)
    wa = wa.at[:D - 2, :16].set(W_pre3).at[D - 2:, 16:].set(W_pre1)
    ba = jnp.concatenate([b_pre3, b_pre1]).reshape(1, 32)
    wb = jnp.zeros((32, 64), f32)
    wb = wb.at[:16, :32].set(W_pre4).at[16:, 32:].set(W_pre2)
    bb = jnp.concatenate([b_pre4, b_pre2]).reshape(1, 64)

    def fold(gb1, gg, gbb, gW2, gb2, bng, bnb):
        a1 = (gg * s).reshape(1, 64)
        c1 = (gb1 * gg * s + gbb).reshape(1, 64)
        sc2 = bng * s
        w2 = gW2 * sc2[None, :]
        b2 = (gb2 * sc2 + bnb).reshape(1, 64)
        return a1, c1, w2, b2

    a10, c10, w20, b20 = fold(gin0_b1, gin0_g, gin0_bb, gin0_W2,
                              gin0_b2, bn0_g, bn0_b)
    a11, c11, w21, b21 = fold(gin1_b1, gin1_g, gin1_bb, gin1_W2,
                              gin1_b2, bn1_g, bn1_b)

    src = edge_index[0]
    dst = edge_index[1]
    pad = EP - E
    src_p = jnp.concatenate([src, jnp.zeros((pad,), jnp.int32)])
    dst_p = jnp.concatenate([dst, jnp.full((pad,), N, jnp.int32)])
    src2 = src_p.reshape(NW, CH, CHW)
    srcidx = jnp.concatenate([src2, src2 + NP], axis=0)
    dstidx = dst_p.reshape(NW, CH, CHW)

    batch3 = jnp.concatenate(
        [batch.astype(jnp.int32), jnp.full((NP - N,), G, jnp.int32)]
    ).reshape(NB, 1, R)

    nx = _pre_call(x, wa, ba, wb, bb)
    agg0 = _seg_call(nx, srcidx, dstidx)
    g0 = _gin0_call(nx, agg0, gin0_W1, a10, c10, w20, b20)
    agg1 = _seg_call(g0, srcidx, dstidx)
    return _fin_call(nx, g0, agg1, batch3, gin1_W1, a11, c11, w21, b21,
                     W_post1, b_post1.reshape(1, 32), W_post2,
                     b_post2.reshape(1, OUT))


# speedup vs baseline: 4.1418x; 4.1418x over previous
import math

import jax
import jax.numpy as jnp
from jax import lax
from jax.experimental import pallas as pl
from jax.experimental.pallas import tpu as pltpu
from jax.experimental.pallas import tpu_sc as plsc

N = 50000
D = 512
E = 800000
G = 512
OUT = 7
_BN_SCALE = 1.0 / math.sqrt(1.0 + 1e-05)

R = 1568
NB = 32
NP = R * NB
CHW = 128
CH = 400
IB = 25
NSL = CH // IB
EPT = CH * CHW
EP = EPT * 16
RPT = NP // 16
RC = RPT // 8


def _pre_body(x_ref, wa_ref, ba_ref, wb_ref, bb_ref, out_ref):
    i = pl.program_id(0)
    xb = x_ref[...]
    t = jnp.maximum(jnp.dot(xb, wa_ref[...], preferred_element_type=jnp.float32)
                    + ba_ref[...], 0.0)
    nx = jnp.maximum(jnp.dot(t, wb_ref[...], preferred_element_type=jnp.float32)
                     + bb_ref[...], 0.0)
    rows = lax.broadcasted_iota(jnp.int32, (R, 64), 0) + i * R
    nx = jnp.where(rows < N, nx, 0.0)
    out_ref[0] = nx[:, :32]
    out_ref[1] = nx[:, 32:]


def _pre_call(x, wa, ba, wb, bb):
    return pl.pallas_call(
        _pre_body,
        grid=(NB,),
        in_specs=[
            pl.BlockSpec((R, D), lambda i: (i, 0)),
            pl.BlockSpec((D, 32), lambda i: (0, 0)),
            pl.BlockSpec((1, 32), lambda i: (0, 0)),
            pl.BlockSpec((32, 64), lambda i: (0, 0)),
            pl.BlockSpec((1, 64), lambda i: (0, 0)),
        ],
        out_specs=pl.BlockSpec((2, R, 32), lambda i: (0, i, 0)),
        out_shape=jax.ShapeDtypeStruct((2, NP, 32), jnp.float32),
    )(x, wa, ba, wb, bb)


def _seg_body(table_hbm, srcidx_hbm, dstidx_hbm, out_hbm,
              src_v, dst_v, rows_v, stage_v, acc_sh, sem):
    c = lax.axis_index("c")
    s = lax.axis_index("s")
    w = c * 16 + s

    zv = jnp.zeros((16,), jnp.float32)

    def _zero(r, carry):
        stage_v[r, pl.ds(0, 16)] = zv
        stage_v[r, pl.ds(16, 16)] = zv
        return carry

    lax.fori_loop(0, RC, _zero, 0)
    base = s * RPT
    for k in range(8):
        pltpu.sync_copy(stage_v, acc_sh.at[pl.ds(base + k * RC, RC)])
    plsc.subcore_barrier()

    def _slab(st, carry):
        pltpu.sync_copy(srcidx_hbm.at[w * NSL + st], src_v)
        pltpu.sync_copy(dstidx_hbm.at[s * NSL + st], dst_v)

        def _edge(j, carry2):
            pltpu.async_copy(table_hbm.at[src_v.at[j]], rows_v, sem).wait()
            pltpu.sync_copy(rows_v, acc_sh.at[dst_v.at[j]], add=True)
            return carry2

        lax.fori_loop(0, IB, _edge, 0)
        return carry

    lax.fori_loop(0, NSL, _slab, 0)
    plsc.subcore_barrier()

    obase = c * NP + base
    for k in range(8):
        pltpu.sync_copy(acc_sh.at[pl.ds(base + k * RC, RC)], stage_v)
        pltpu.sync_copy(stage_v, out_hbm.at[pl.ds(obase + k * RC, RC)])


def _seg_call(table2, srcidx, dstidx):
    table_flat = table2.reshape(2 * NP, 32)
    mesh = plsc.VectorSubcoreMesh(core_axis_name="c", subcore_axis_name="s",
                                  num_cores=2, num_subcores=16)
    out = pl.kernel(
        _seg_body,
        out_type=jax.ShapeDtypeStruct((2 * NP, 32), jnp.float32),
        mesh=mesh,
        scratch_types=[
            pltpu.VMEM((IB, CHW), jnp.int32),
            pltpu.VMEM((IB, CHW), jnp.int32),
            pltpu.VMEM((CHW, 32), jnp.float32),
            pltpu.VMEM((RC, 32), jnp.float32),
            pltpu.VMEM_SHARED((NP, 32), jnp.float32),
            pltpu.SemaphoreType.DMA,
        ],
        compiler_params=pltpu.CompilerParams(use_tc_tiling_on_sc=False),
    )(table_flat, srcidx, dstidx)
    return out.reshape(2, NP, 32)


def _gin0_body(nx_ref, agg_ref, w1_ref, a1_ref, c1_ref, w2_ref, b2_ref,
               out_ref):
    nxb = jnp.concatenate([nx_ref[0], nx_ref[1]], axis=1)
    aggb = jnp.concatenate([agg_ref[0], agg_ref[1]], axis=1)
    h = nxb + aggb
    t = jnp.maximum(jnp.dot(h, w1_ref[...], preferred_element_type=jnp.float32)
                    * a1_ref[...] + c1_ref[...], 0.0)
    g0 = (jnp.dot(t, w2_ref[...], preferred_element_type=jnp.float32)
          + b2_ref[...] + nxb)
    out_ref[0] = g0[:, :32]
    out_ref[1] = g0[:, 32:]


def _gin0_call(nx, agg, w1, a1, c1, w2, b2):
    spec2 = pl.BlockSpec((2, R, 32), lambda i: (0, i, 0))
    return pl.pallas_call(
        _gin0_body,
        grid=(NB,),
        in_specs=[
            spec2, spec2,
            pl.BlockSpec((64, 64), lambda i: (0, 0)),
            pl.BlockSpec((1, 64), lambda i: (0, 0)),
            pl.BlockSpec((1, 64), lambda i: (0, 0)),
            pl.BlockSpec((64, 64), lambda i: (0, 0)),
            pl.BlockSpec((1, 64), lambda i: (0, 0)),
        ],
        out_specs=spec2,
        out_shape=jax.ShapeDtypeStruct((2, NP, 32), jnp.float32),
    )(nx, agg, w1, a1, c1, w2, b2)


def _fin_body(nx_ref, g0_ref, agg_ref, batch_ref, w1_ref, a1_ref, c1_ref,
              w2_ref, b2_ref, wp1_ref, bp1_ref, wp2_ref, bp2_ref,
              out_ref, acc_ref):
    i = pl.program_id(0)
    nxb = jnp.concatenate([nx_ref[0], nx_ref[1]], axis=1)
    g0b = jnp.concatenate([g0_ref[0], g0_ref[1]], axis=1)
    aggb = jnp.concatenate([agg_ref[0], agg_ref[1]], axis=1)
    h = g0b + aggb
    t = jnp.maximum(jnp.dot(h, w1_ref[...], preferred_element_type=jnp.float32)
                    * a1_ref[...] + c1_ref[...], 0.0)
    g1 = (jnp.dot(t, w2_ref[...], preferred_element_type=jnp.float32)
          + b2_ref[...] + g0b + nxb)

    bb = batch_ref[0]
    onehot = (lax.broadcasted_iota(jnp.int32, (G, R), 0) == bb).astype(
        jnp.float32)
    gpad = jnp.concatenate(
        [g1, jnp.ones((R, 1), jnp.float32), jnp.zeros((R, 63), jnp.float32)],
        axis=1)
    contrib = jnp.dot(onehot, gpad, preferred_element_type=jnp.float32)

    @pl.when(i == 0)
    def _():
        acc_ref[...] = jnp.zeros((G, 128), jnp.float32)

    acc_ref[...] += contrib

    @pl.when(i == NB - 1)
    def _():
        a = acc_ref[...]
        mean = a[:, :64] / jnp.maximum(a[:, 64:65], 1.0)
        t2 = jnp.maximum(
            jnp.dot(mean, wp1_ref[...], preferred_element_type=jnp.float32)
            + bp1_ref[...], 0.0)
        o = (jnp.dot(t2, wp2_ref[...], preferred_element_type=jnp.float32)
             + bp2_ref[...])
        m = jnp.max(o, axis=1, keepdims=True)
        lse = m + jnp.log(jnp.sum(jnp.exp(o - m), axis=1, keepdims=True))
        out_ref[...] = o - lse


def _fin_call(nx, g0, agg, batch3, w1, a1, c1, w2, b2, wp1, bp1, wp2, bp2):
    spec2 = pl.BlockSpec((2, R, 32), lambda i: (0, i, 0))
    wfull = lambda r, c: pl.BlockSpec((r, c), lambda i: (0, 0))
    return pl.pallas_call(
        _fin_body,
        grid=(NB,),
        in_specs=[
            spec2, spec2, spec2,
            pl.BlockSpec((1, 1, R), lambda i: (i, 0, 0)),
            wfull(64, 64), wfull(1, 64), wfull(1, 64), wfull(64, 64),
            wfull(1, 64), wfull(64, 32), wfull(1, 32), wfull(32, OUT),
            wfull(1, OUT),
        ],
        out_specs=pl.BlockSpec((G, OUT), lambda i: (0, 0)),
        out_shape=jax.ShapeDtypeStruct((G, OUT), jnp.float32),
        scratch_shapes=[pltpu.VMEM((G, 128), jnp.float32)],
    )(nx, g0, agg, batch3, w1, a1, c1, w2, b2, wp1, bp1, wp2, bp2)


def kernel(x, edge_index, batch,
           W_pre1, b_pre1, W_pre2, b_pre2, W_pre3, b_pre3, W_pre4, b_pre4,
           gin0_W1, gin0_b1, gin0_g, gin0_bb, gin0_W2, gin0_b2, bn0_g, bn0_b,
           gin1_W1, gin1_b1, gin1_g, gin1_bb, gin1_W2, gin1_b2, bn1_g, bn1_b,
           W_post1, b_post1, W_post2, b_post2):
    f32 = jnp.float32
    s = _BN_SCALE

    wa = jnp.zeros((D, 32), f32)
    wa = wa.at[:D - 2, :16].set(W_pre3).at[D - 2:, 16:].set(W_pre1)
    ba = jnp.concatenate([b_pre3, b_pre1]).reshape(1, 32)
    wb = jnp.zeros((32, 64), f32)
    wb = wb.at[:16, :32].set(W_pre4).at[16:, 32:].set(W_pre2)
    bb = jnp.concatenate([b_pre4, b_pre2]).reshape(1, 64)

    def fold(gb1, gg, gbb, gW2, gb2, bng, bnb):
        a1 = (gg * s).reshape(1, 64)
        c1 = (gb1 * gg * s + gbb).reshape(1, 64)
        sc2 = bng * s
        w2 = gW2 * sc2[None, :]
        b2 = (gb2 * sc2 + bnb).reshape(1, 64)
        return a1, c1, w2, b2

    a10, c10, w20, b20 = fold(gin0_b1, gin0_g, gin0_bb, gin0_W2,
                              gin0_b2, bn0_g, bn0_b)
    a11, c11, w21, b21 = fold(gin1_b1, gin1_g, gin1_bb, gin1_W2,
                              gin1_b2, bn1_g, bn1_b)

    src = edge_index[0]
    dst = edge_index[1]
    pad = EP - E
    src_p = jnp.concatenate([src, jnp.zeros((pad,), jnp.int32)])
    dst_p = jnp.concatenate([dst, jnp.full((pad,), N, jnp.int32)])
    src2 = src_p.reshape(16, NSL, IB, CHW)
    srcidx = jnp.concatenate([src2, src2 + NP], axis=0).reshape(
        32 * NSL, IB, CHW)
    dstidx = dst_p.reshape(16 * NSL, IB, CHW)

    batch3 = jnp.concatenate(
        [batch.astype(jnp.int32), jnp.full((NP - N,), G, jnp.int32)]
    ).reshape(NB, 1, R)

    nx = _pre_call(x, wa, ba, wb, bb)
    agg0 = _seg_call(nx, srcidx, dstidx)
    g0 = _gin0_call(nx, agg0, gin0_W1, a10, c10, w20, b20)
    agg1 = _seg_call(g0, srcidx, dstidx)
    return _fin_call(nx, g0, agg1, batch3, gin1_W1, a11, c11, w21, b21,
                     W_post1, b_post1.reshape(1, 32), W_post2,
                     b_post2.reshape(1, OUT))


# speedup vs baseline: 5.8057x; 1.4017x over previous
import math

import jax
import jax.numpy as jnp
from jax import lax
from jax.experimental import pallas as pl
from jax.experimental.pallas import tpu as pltpu
from jax.experimental.pallas import tpu_sc as plsc

N = 50000
D = 512
E = 800000
G = 512
OUT = 7
_BN_SCALE = 1.0 / math.sqrt(1.0 + 1e-05)

R = 1568
NB = 32
NP = R * NB
CHW = 128
CH = 400
IB = 25
NBUF = 5
NSL = CH // IB
EPT = CH * CHW
EP = EPT * 16
RPT = NP // 16
RC = RPT // 8


def _pre_body(x_ref, wa_ref, ba_ref, wb_ref, bb_ref, out_ref):
    i = pl.program_id(0)
    xb = x_ref[...]
    t = jnp.maximum(jnp.dot(xb, wa_ref[...], preferred_element_type=jnp.float32)
                    + ba_ref[...], 0.0)
    nx = jnp.maximum(jnp.dot(t, wb_ref[...], preferred_element_type=jnp.float32)
                     + bb_ref[...], 0.0)
    rows = lax.broadcasted_iota(jnp.int32, (R, 64), 0) + i * R
    nx = jnp.where(rows < N, nx, 0.0)
    out_ref[0] = nx[:, :32]
    out_ref[1] = nx[:, 32:]


def _pre_call(x, wa, ba, wb, bb):
    return pl.pallas_call(
        _pre_body,
        grid=(NB,),
        in_specs=[
            pl.BlockSpec((R, D), lambda i: (i, 0)),
            pl.BlockSpec((D, 32), lambda i: (0, 0)),
            pl.BlockSpec((1, 32), lambda i: (0, 0)),
            pl.BlockSpec((32, 64), lambda i: (0, 0)),
            pl.BlockSpec((1, 64), lambda i: (0, 0)),
        ],
        out_specs=pl.BlockSpec((2, R, 32), lambda i: (0, i, 0)),
        out_shape=jax.ShapeDtypeStruct((2, NP, 32), jnp.float32),
    )(x, wa, ba, wb, bb)


def _seg_body(table_hbm, srcidx_hbm, dstidx_hbm, zeros_hbm, out_hbm,
              src_v, dst_v, r0, r1, r2, r3, r4, acc_sh,
              g0, g1, g2, g3, g4):
    c = lax.axis_index("c")
    s = lax.axis_index("s")
    w = c * 16 + s
    rows = (r0, r1, r2, r3, r4)
    gsem = (g0, g1, g2, g3, g4)

    base = s * RPT
    pltpu.sync_copy(zeros_hbm, acc_sh.at[pl.ds(base, RPT)])
    plsc.subcore_barrier()

    def _slab(st, carry):
        pltpu.sync_copy(srcidx_hbm.at[w * NSL + st], src_v)
        pltpu.sync_copy(dstidx_hbm.at[s * NSL + st], dst_v)
        for b in range(NBUF):
            pltpu.async_copy(table_hbm.at[src_v.at[b]], rows[b], gsem[b])

        def _grp(g, carry2):
            for b in range(NBUF):
                j = g * NBUF + b
                pltpu.make_async_copy(table_hbm.at[src_v.at[j]], rows[b],
                                      gsem[b]).wait()
                pltpu.sync_copy(rows[b], acc_sh.at[dst_v.at[j]], add=True)

                @pl.when(g < IB // NBUF - 1)
                def _():
                    pltpu.async_copy(table_hbm.at[src_v.at[j + NBUF]],
                                     rows[b], gsem[b])
            return carry2

        lax.fori_loop(0, IB // NBUF, _grp, 0)
        return carry

    lax.fori_loop(0, NSL, _slab, 0)
    plsc.subcore_barrier()

    pltpu.sync_copy(acc_sh.at[pl.ds(base, RPT)],
                    out_hbm.at[pl.ds(c * NP + base, RPT)])


def _seg_call(table2, srcidx, dstidx, zeros):
    table_flat = table2.reshape(2 * NP, 32)
    mesh = plsc.VectorSubcoreMesh(core_axis_name="c", subcore_axis_name="s",
                                  num_cores=2, num_subcores=16)
    out = pl.kernel(
        _seg_body,
        out_type=jax.ShapeDtypeStruct((2 * NP, 32), jnp.float32),
        mesh=mesh,
        scratch_types=(
            [pltpu.VMEM((IB, CHW), jnp.int32),
             pltpu.VMEM((IB, CHW), jnp.int32)]
            + [pltpu.VMEM((CHW, 32), jnp.float32) for _ in range(NBUF)]
            + [pltpu.VMEM_SHARED((NP, 32), jnp.float32)]
            + [pltpu.SemaphoreType.DMA for _ in range(NBUF)]
        ),
        compiler_params=pltpu.CompilerParams(use_tc_tiling_on_sc=False),
    )(table_flat, srcidx, dstidx, zeros)
    return out.reshape(2, NP, 32)


def _gin0_body(nx_ref, agg_ref, w1_ref, a1_ref, c1_ref, w2_ref, b2_ref,
               out_ref):
    nxb = jnp.concatenate([nx_ref[0], nx_ref[1]], axis=1)
    aggb = jnp.concatenate([agg_ref[0], agg_ref[1]], axis=1)
    h = nxb + aggb
    t = jnp.maximum(jnp.dot(h, w1_ref[...], preferred_element_type=jnp.float32)
                    * a1_ref[...] + c1_ref[...], 0.0)
    g0 = (jnp.dot(t, w2_ref[...], preferred_element_type=jnp.float32)
          + b2_ref[...] + nxb)
    out_ref[0] = g0[:, :32]
    out_ref[1] = g0[:, 32:]


def _gin0_call(nx, agg, w1, a1, c1, w2, b2):
    spec2 = pl.BlockSpec((2, R, 32), lambda i: (0, i, 0))
    return pl.pallas_call(
        _gin0_body,
        grid=(NB,),
        in_specs=[
            spec2, spec2,
            pl.BlockSpec((64, 64), lambda i: (0, 0)),
            pl.BlockSpec((1, 64), lambda i: (0, 0)),
            pl.BlockSpec((1, 64), lambda i: (0, 0)),
            pl.BlockSpec((64, 64), lambda i: (0, 0)),
            pl.BlockSpec((1, 64), lambda i: (0, 0)),
        ],
        out_specs=spec2,
        out_shape=jax.ShapeDtypeStruct((2, NP, 32), jnp.float32),
    )(nx, agg, w1, a1, c1, w2, b2)


def _fin_body(nx_ref, g0_ref, agg_ref, batch_ref, w1_ref, a1_ref, c1_ref,
              w2_ref, b2_ref, wp1_ref, bp1_ref, wp2_ref, bp2_ref,
              out_ref, acc_ref):
    i = pl.program_id(0)
    nxb = jnp.concatenate([nx_ref[0], nx_ref[1]], axis=1)
    g0b = jnp.concatenate([g0_ref[0], g0_ref[1]], axis=1)
    aggb = jnp.concatenate([agg_ref[0], agg_ref[1]], axis=1)
    h = g0b + aggb
    t = jnp.maximum(jnp.dot(h, w1_ref[...], preferred_element_type=jnp.float32)
                    * a1_ref[...] + c1_ref[...], 0.0)
    g1 = (jnp.dot(t, w2_ref[...], preferred_element_type=jnp.float32)
          + b2_ref[...] + g0b + nxb)

    bb = batch_ref[0]
    onehot = (lax.broadcasted_iota(jnp.int32, (G, R), 0) == bb).astype(
        jnp.float32)
    gpad = jnp.concatenate(
        [g1, jnp.ones((R, 1), jnp.float32), jnp.zeros((R, 63), jnp.float32)],
        axis=1)
    contrib = jnp.dot(onehot, gpad, preferred_element_type=jnp.float32)

    @pl.when(i == 0)
    def _():
        acc_ref[...] = jnp.zeros((G, 128), jnp.float32)

    acc_ref[...] += contrib

    @pl.when(i == NB - 1)
    def _():
        a = acc_ref[...]
        mean = a[:, :64] / jnp.maximum(a[:, 64:65], 1.0)
        t2 = jnp.maximum(
            jnp.dot(mean, wp1_ref[...], preferred_element_type=jnp.float32)
            + bp1_ref[...], 0.0)
        o = (jnp.dot(t2, wp2_ref[...], preferred_element_type=jnp.float32)
             + bp2_ref[...])
        m = jnp.max(o, axis=1, keepdims=True)
        lse = m + jnp.log(jnp.sum(jnp.exp(o - m), axis=1, keepdims=True))
        out_ref[...] = o - lse


def _fin_call(nx, g0, agg, batch3, w1, a1, c1, w2, b2, wp1, bp1, wp2, bp2):
    spec2 = pl.BlockSpec((2, R, 32), lambda i: (0, i, 0))
    wfull = lambda r, c: pl.BlockSpec((r, c), lambda i: (0, 0))
    return pl.pallas_call(
        _fin_body,
        grid=(NB,),
        in_specs=[
            spec2, spec2, spec2,
            pl.BlockSpec((1, 1, R), lambda i: (i, 0, 0)),
            wfull(64, 64), wfull(1, 64), wfull(1, 64), wfull(64, 64),
            wfull(1, 64), wfull(64, 32), wfull(1, 32), wfull(32, OUT),
            wfull(1, OUT),
        ],
        out_specs=pl.BlockSpec((G, OUT), lambda i: (0, 0)),
        out_shape=jax.ShapeDtypeStruct((G, OUT), jnp.float32),
        scratch_shapes=[pltpu.VMEM((G, 128), jnp.float32)],
    )(nx, g0, agg, batch3, w1, a1, c1, w2, b2, wp1, bp1, wp2, bp2)


def kernel(x, edge_index, batch,
           W_pre1, b_pre1, W_pre2, b_pre2, W_pre3, b_pre3, W_pre4, b_pre4,
           gin0_W1, gin0_b1, gin0_g, gin0_bb, gin0_W2, gin0_b2, bn0_g, bn0_b,
           gin1_W1, gin1_b1, gin1_g, gin1_bb, gin1_W2, gin1_b2, bn1_g, bn1_b,
           W_post1, b_post1, W_post2, b_post2):
    f32 = jnp.float32
    s = _BN_SCALE

    wa = jnp.zeros((D, 32), f32)
    wa = wa.at[:D - 2, :16].set(W_pre3).at[D - 2:, 16:].set(W_pre1)
    ba = jnp.concatenate([b_pre3, b_pre1]).reshape(1, 32)
    wb = jnp.zeros((32, 64), f32)
    wb = wb.at[:16, :32].set(W_pre4).at[16:, 32:].set(W_pre2)
    bb = jnp.concatenate([b_pre4, b_pre2]).reshape(1, 64)

    def fold(gb1, gg, gbb, gW2, gb2, bng, bnb):
        a1 = (gg * s).reshape(1, 64)
        c1 = (gb1 * gg * s + gbb).reshape(1, 64)
        sc2 = bng * s
        w2 = gW2 * sc2[None, :]
        b2 = (gb2 * sc2 + bnb).reshape(1, 64)
        return a1, c1, w2, b2

    a10, c10, w20, b20 = fold(gin0_b1, gin0_g, gin0_bb, gin0_W2,
                              gin0_b2, bn0_g, bn0_b)
    a11, c11, w21, b21 = fold(gin1_b1, gin1_g, gin1_bb, gin1_W2,
                              gin1_b2, bn1_g, bn1_b)

    src = edge_index[0]
    dst = edge_index[1]
    pad = EP - E
    src_p = jnp.concatenate([src, jnp.zeros((pad,), jnp.int32)])
    dst_p = jnp.concatenate([dst, jnp.full((pad,), N, jnp.int32)])
    src2 = src_p.reshape(16, NSL, IB, CHW)
    srcidx = jnp.concatenate([src2, src2 + NP], axis=0).reshape(
        32 * NSL, IB, CHW)
    dstidx = dst_p.reshape(16 * NSL, IB, CHW)

    batch3 = jnp.concatenate(
        [batch.astype(jnp.int32), jnp.full((NP - N,), G, jnp.int32)]
    ).reshape(NB, 1, R)

    zeros = jnp.zeros((RPT, 32), jnp.float32)
    nx = _pre_call(x, wa, ba, wb, bb)
    agg0 = _seg_call(nx, srcidx, dstidx, zeros)
    g0 = _gin0_call(nx, agg0, gin0_W1, a10, c10, w20, b20)
    agg1 = _seg_call(g0, srcidx, dstidx, zeros)
    return _fin_call(nx, g0, agg1, batch3, gin1_W1, a11, c11, w21, b21,
                     W_post1, b_post1.reshape(1, 32), W_post2,
                     b_post2.reshape(1, OUT))
